# baseline (speedup 1.0000x reference)
import functools

import jax
import jax.numpy as jnp
from jax import lax
from jax.experimental import pallas as pl
from jax.experimental.pallas import tpu as pltpu
from jax.experimental.pallas import tpu_sc as plsc

NSUB = 16
NCORE = 2
NW = NSUB * NCORE
CHUNK = 128
HROWS = 128
DUMMY = 112
VG = CHUNK // 16


def _prop_body(cpt, p, tbl, pk3, outp, pk_v, sidx, didx, rows0, rows1,
               acc_sh, sem0, sem1):
    assert cpt % 2 == 1
    c = lax.axis_index("c")
    s = lax.axis_index("s")
    wid = c * NSUB + s
    rpt = p // NSUB

    pltpu.sync_copy(pk3.at[wid], pk_v)

    def unpack_src(j, slot):
        for kk in range(VG):
            pk = pk_v[j, pl.ds(16 * kk, 16)]
            sidx[slot, pl.ds(16 * kk, 16)] = lax.bitwise_and(pk, 0xFFFF)

    def unpack_dst(j):
        for kk in range(VG):
            pk = pk_v[j, pl.ds(16 * kk, 16)]
            didx[0, pl.ds(16 * kk, 16)] = lax.shift_right_logical(pk, 16)

    z = jnp.zeros((16,), jnp.float32)

    def zrow(i, carry):
        for kk in range(8):
            rows0[i, pl.ds(16 * kk, 16)] = z
        return carry

    lax.fori_loop(0, CHUNK, zrow, 0)
    nfull, nrem = rpt // CHUNK, rpt % CHUNK
    zcopies = [(j * CHUNK, CHUNK) for j in range(nfull)]
    if nrem:
        zcopies.append((nfull * CHUNK, nrem))
    for off, nr in zcopies:
        pltpu.async_copy(rows0.at[pl.ds(0, nr)],
                         acc_sh.at[pl.ds(s * rpt + off, nr)], sem0)
    for off, nr in zcopies:
        pltpu.make_async_copy(rows0.at[pl.ds(0, nr)],
                              acc_sh.at[pl.ds(s * rpt + off, nr)],
                              sem0).wait()
    plsc.subcore_barrier()

    unpack_src(0, 0)
    pltpu.async_copy(tbl.at[sidx.at[0]], rows0, sem0)
    unpack_src(1, 1)
    pltpu.async_copy(tbl.at[sidx.at[1]], rows1, sem1)

    def body(k, carry):
        for t, (rw, sm) in enumerate(((rows0, sem0), (rows1, sem1))):
            j = 2 * k + t
            pltpu.make_async_copy(tbl.at[sidx.at[t]], rw, sm).wait()
            unpack_dst(j)
            pltpu.sync_copy(rw, acc_sh.at[didx.at[0]], add=True)

            def _refill():
                unpack_src(j + 2, t)
                pltpu.async_copy(tbl.at[sidx.at[t]], rw, sm)

            if t == 0:
                _refill()
            else:
                pl.when(j + 2 < cpt)(_refill)
        return carry

    lax.fori_loop(0, cpt // 2, body, 0)
    pltpu.make_async_copy(tbl.at[sidx.at[0]], rows0, sem0).wait()
    unpack_dst(cpt - 1)
    pltpu.sync_copy(rows0, acc_sh.at[didx.at[0]], add=True)

    plsc.subcore_barrier()
    pltpu.sync_copy(acc_sh.at[pl.ds(s * rpt, rpt)],
                    outp.at[c, pl.ds(s * rpt, rpt)])


def _deg_body(cpt, p, pk3, degp, pk_v, hist_v, idx_v, deg_sh):
    c = lax.axis_index("c")
    s = lax.axis_index("s")
    wid = c * NSUB + s
    pr = HROWS

    pltpu.sync_copy(pk3.at[wid], pk_v)
    zero = jnp.zeros((16,), jnp.float32)

    def zrow(i, carry):
        for kk in range(8):
            hist_v[i, pl.ds(16 * kk, 16)] = zero
        return carry

    lax.fori_loop(0, pr, zrow, 0)
    for i in range(8):
        idx_v[0, pl.ds(16 * i, 16)] = lax.iota(jnp.int32, 16) + 16 * i
    rps = pr // NSUB
    pltpu.sync_copy(hist_v.at[pl.ds(0, rps)], deg_sh.at[pl.ds(s * rps, rps)])
    plsc.subcore_barrier()

    one = jnp.ones((16,), jnp.float32)

    def body(j, carry):
        for kk in range(VG):
            pk = pk_v[j, pl.ds(16 * kk, 16)]
            dd = lax.shift_right_logical(pk, 16)
            row = lax.shift_right_logical(dd, 7)
            col = lax.bitwise_and(dd, 127)
            plsc.addupdate_scatter(hist_v, [row, col], one)
        return carry

    lax.fori_loop(0, cpt, body, 0)

    pltpu.sync_copy(hist_v, deg_sh.at[idx_v.at[0]], add=True)
    plsc.subcore_barrier()
    pltpu.sync_copy(deg_sh.at[pl.ds(s * rps, rps)],
                    degp.at[c, pl.ds(s * rps, rps)])


def _dis_rows(degp_ref, p):
    deg = (degp_ref[0] + degp_ref[1]).reshape(-1)[:p]
    return jnp.where(deg > 0.0, lax.rsqrt(jnp.maximum(deg, 1e-12)), 0.0)


def _scale_mm_body(x_ref, w_ref, degp_ref, o_ref):
    dis = _dis_rows(degp_ref, x_ref.shape[0])
    xw = jnp.dot(x_ref[...], w_ref[...], preferred_element_type=jnp.float32)
    o_ref[...] = xw * dis[:, None]


def _relu_scale_body(p_ref, degp_ref, b_ref, o_ref):
    dis = _dis_rows(degp_ref, p_ref.shape[1])
    t = (p_ref[0] + p_ref[1]) * dis[:, None]
    h = jnp.maximum(t + b_ref[...], 0.0)
    o_ref[...] = h * dis[:, None]


def _final_mm_body(q_ref, degp_ref, w_ref, b_ref, o_ref):
    dis = _dis_rows(degp_ref, q_ref.shape[1])
    t = (q_ref[0] + q_ref[1]) * dis[:, None]
    o_ref[...] = jnp.dot(t, w_ref[...],
                         preferred_element_type=jnp.float32) + b_ref[...]


def kernel(x, edge_index, W1, b1, W_mu, b_mu, W_lv, b_lv):
    n = x.shape[0]
    d_in = x.shape[1]
    d_hid = W1.shape[1]
    d_out = W_mu.shape[1]
    e = edge_index.shape[1]

    p = n + DUMMY
    assert p % 128 == 0 and p < (1 << 16)
    ne = e + n
    epg = NW * CHUNK
    ep = ((ne + epg - 1) // epg) * epg
    cpt = ep // epg
    assert cpt % 2 == 1
    npad = ep - ne

    ei = edge_index.astype(jnp.int32)
    loop = jnp.arange(n, dtype=jnp.int32)
    padi = jnp.arange(npad, dtype=jnp.int32)
    src = jnp.concatenate([ei[0], loop, padi % n])
    dst = jnp.concatenate([ei[1], loop, n + (padi % DUMMY)])
    pk3 = ((dst << 16) | src).reshape(NW, cpt, CHUNK)

    xpad = jnp.pad(x, ((0, p - n), (0, 0)))

    mesh = plsc.VectorSubcoreMesh(core_axis_name="c", subcore_axis_name="s")
    sc_params = pltpu.CompilerParams(needs_layout_passes=False)

    assert HROWS * 128 >= p and HROWS % NSUB == 0
    deg_call = pl.kernel(
        functools.partial(_deg_body, cpt, p),
        out_type=jax.ShapeDtypeStruct((NCORE, HROWS, 128), jnp.float32),
        mesh=mesh,
        compiler_params=sc_params,
        scratch_types=[
            pltpu.VMEM((cpt, CHUNK), jnp.int32),
            pltpu.VMEM((HROWS, 128), jnp.float32),
            pltpu.VMEM((1, CHUNK), jnp.int32),
            pltpu.VMEM_SHARED((HROWS, 128), jnp.float32),
        ],
    )
    degp = deg_call(pk3)

    prop_call = pl.kernel(
        functools.partial(_prop_body, cpt, p),
        out_type=jax.ShapeDtypeStruct((NCORE, p, d_hid), jnp.float32),
        mesh=mesh,
        compiler_params=sc_params,
        scratch_types=[
            pltpu.VMEM((cpt, CHUNK), jnp.int32),
            pltpu.VMEM((2, CHUNK), jnp.int32),
            pltpu.VMEM((1, CHUNK), jnp.int32),
            pltpu.VMEM((CHUNK, d_hid), jnp.float32),
            pltpu.VMEM((CHUNK, d_hid), jnp.float32),
            pltpu.VMEM_SHARED((p, d_hid), jnp.float32),
            pltpu.SemaphoreType.DMA,
            pltpu.SemaphoreType.DMA,
        ],
    )

    grid = (1,)
    row_spec = pl.BlockSpec((p, d_hid), lambda i: (0, 0))
    degp_spec = pl.BlockSpec((NCORE, HROWS, 128), lambda i: (0, 0, 0))
    part_spec = pl.BlockSpec((NCORE, p, d_hid), lambda i: (0, 0, 0))
    w_spec = pl.BlockSpec((d_in, d_hid), lambda i: (0, 0))
    b_spec = pl.BlockSpec((1, d_hid), lambda i: (0, 0))

    ysc = pl.pallas_call(
        _scale_mm_body,
        grid=grid,
        in_specs=[row_spec, w_spec, degp_spec],
        out_specs=row_spec,
        out_shape=jax.ShapeDtypeStruct((p, d_hid), jnp.float32),
    )(xpad, W1, degp)

    pp = prop_call(ysc, pk3)

    hs = pl.pallas_call(
        _relu_scale_body,
        grid=grid,
        in_specs=[part_spec, degp_spec, b_spec],
        out_specs=row_spec,
        out_shape=jax.ShapeDtypeStruct((p, d_hid), jnp.float32),
    )(pp, degp, b1.reshape(1, d_hid))

    qq = prop_call(hs, pk3)

    wcat = jnp.concatenate([W_mu, W_lv], axis=1)
    bcat = jnp.concatenate([b_mu, b_lv]).reshape(1, 2 * d_out)
    wcat_spec = pl.BlockSpec((d_hid, 2 * d_out), lambda i: (0, 0))
    bcat_spec = pl.BlockSpec((1, 2 * d_out), lambda i: (0, 0))
    out = pl.pallas_call(
        _final_mm_body,
        grid=grid,
        in_specs=[part_spec, degp_spec, wcat_spec, bcat_spec],
        out_specs=pl.BlockSpec((p, 2 * d_out), lambda i: (0, 0)),
        out_shape=jax.ShapeDtypeStruct((p, 2 * d_out), jnp.float32),
    )(qq, degp, wcat, bcat)

    mu = out[:n, :d_out]
    lv = out[:n, d_out:]
    return (mu, lv)
